# baseline (speedup 1.0000x reference)
import functools

import jax
import jax.numpy as jnp
from jax import lax
from jax.experimental import pallas as pl
from jax.experimental.pallas import tpu as pltpu
from jax.experimental.pallas import tpu_sc as plsc

B = 4096
HALF = B // 2
LANES = 128
QROWS = 784
VPAD = QROWS * LANES
BR = 512


def _sc_gather(ids, prob_table):
    info = plsc.get_sparse_core_info()
    nw = info.num_cores * info.num_subcores
    per_w = HALF // nw
    mesh = plsc.VectorSubcoreMesh(core_axis_name="c", subcore_axis_name="s")

    @functools.partial(
        pl.kernel,
        mesh=mesh,
        out_type=jax.ShapeDtypeStruct((HALF,), jnp.float32),
        scratch_types=[
            pltpu.VMEM((per_w,), jnp.int32),
            pltpu.VMEM((per_w,), jnp.float32),
            pltpu.SemaphoreType.DMA,
        ],
    )
    def gather_kernel(idx_hbm, table_hbm, out_hbm, idx_v, rows_v, sem):
        wid = lax.axis_index("s") * info.num_cores + lax.axis_index("c")
        base = wid * per_w
        pltpu.sync_copy(idx_hbm.at[pl.ds(base, per_w)], idx_v)
        pltpu.async_copy(table_hbm.at[idx_v], rows_v, sem).wait()
        pltpu.sync_copy(rows_v, out_hbm.at[pl.ds(base, per_w)])

    return gather_kernel(ids, prob_table)


def _tc_left(logits, ids_left, table_t):
    def body(ids_ref, table_ref, logits_ref, out_ref, corr_ref):
        @pl.when(pl.program_id(0) == 0)
        def _():
            q = ids_ref[...] // LANES
            r = ids_ref[...] % LANES
            iota_q = lax.broadcasted_iota(jnp.int32, (QROWS, HALF), 0)
            onehot_q = (iota_q == q).astype(jnp.float32)
            rows = lax.dot_general(
                table_ref[...], onehot_q,
                (((1,), (0,)), ((), ())),
                preferred_element_type=jnp.float32,
                precision=lax.Precision.HIGHEST)
            iota_r = lax.broadcasted_iota(jnp.int32, (LANES, HALF), 0)
            vals = jnp.sum(
                jnp.where(iota_r == r, rows, 0.0), axis=0, keepdims=True)
            corr_ref[...] = jnp.log(vals)

        out_ref[...] = logits_ref[...] - corr_ref[...]

    return pl.pallas_call(
        body,
        grid=(B // BR,),
        in_specs=[
            pl.BlockSpec((1, HALF), lambda i: (0, 0)),
            pl.BlockSpec((LANES, QROWS), lambda i: (0, 0)),
            pl.BlockSpec((BR, HALF), lambda i: (i, 0)),
        ],
        out_specs=pl.BlockSpec((BR, HALF), lambda i: (i, 0)),
        out_shape=jax.ShapeDtypeStruct((B, B), jnp.float32),
        scratch_shapes=[pltpu.VMEM((1, HALF), jnp.float32)],
    )(ids_left, table_t, logits)


def _tc_right(buf, logits, probs_right):
    def body(buf_ref, probs_ref, logits_ref, out_ref, corr_ref):
        @pl.when(pl.program_id(0) == 0)
        def _():
            corr_ref[...] = jnp.log(probs_ref[...])

        out_ref[...] = logits_ref[...] - corr_ref[...]

    return pl.pallas_call(
        body,
        grid=(B // BR,),
        in_specs=[
            pl.BlockSpec(memory_space=pl.ANY),
            pl.BlockSpec((1, HALF), lambda i: (0, 0)),
            pl.BlockSpec((BR, HALF), lambda i: (i, 1)),
        ],
        out_specs=pl.BlockSpec((BR, HALF), lambda i: (i, 1)),
        out_shape=jax.ShapeDtypeStruct((B, B), jnp.float32),
        scratch_shapes=[pltpu.VMEM((1, HALF), jnp.float32)],
        input_output_aliases={0: 0},
    )(buf, probs_right, logits)


def kernel(logits, candidate_ids, prob_table):
    ids = candidate_ids.reshape(-1).astype(jnp.int32)
    probs_right = _sc_gather(ids[HALF:], prob_table)
    table_t = jnp.pad(prob_table, (0, VPAD - prob_table.shape[0])).reshape(
        QROWS, LANES).T
    part = _tc_left(logits, ids[:HALF].reshape(1, HALF), table_t)
    return _tc_right(part, logits, probs_right.reshape(1, HALF))


# speedup vs baseline: 1.1347x; 1.1347x over previous
import functools

import jax
import jax.numpy as jnp
from jax import lax
from jax.experimental import pallas as pl
from jax.experimental.pallas import tpu as pltpu
from jax.experimental.pallas import tpu_sc as plsc

B = 4096


def _sc_gather(ids, prob_table):
    info = plsc.get_sparse_core_info()
    _NC, _NS = 1, info.num_subcores
    _B_PER_W = B // (_NC * _NS)
    mesh = plsc.VectorSubcoreMesh(
        core_axis_name="c", subcore_axis_name="s", num_cores=1)

    @functools.partial(
        pl.kernel,
        mesh=mesh,
        out_type=jax.ShapeDtypeStruct((B,), jnp.float32),
        scratch_types=[
            pltpu.VMEM((_B_PER_W,), jnp.int32),
            pltpu.VMEM((_B_PER_W,), jnp.float32),
            pltpu.SemaphoreType.DMA,
        ],
    )
    def gather_kernel(idx_hbm, table_hbm, out_hbm, idx_v, rows_v, sem):
        wid = lax.axis_index("s") * _NC + lax.axis_index("c")
        base = wid * _B_PER_W
        pltpu.sync_copy(idx_hbm.at[pl.ds(base, _B_PER_W)], idx_v)
        pltpu.async_copy(table_hbm.at[idx_v], rows_v, sem).wait()
        pltpu.sync_copy(rows_v, out_hbm.at[pl.ds(base, _B_PER_W)])

    return gather_kernel(ids, prob_table)


def _tc_subtract(logits, probs_row, block_rows=512):
    def body(probs_ref, logits_ref, out_ref, corr_ref):
        @pl.when(pl.program_id(0) == 0)
        def _():
            corr_ref[...] = jnp.log(probs_ref[...])

        out_ref[...] = logits_ref[...] - corr_ref[...]

    return pl.pallas_call(
        body,
        grid=(B // block_rows,),
        in_specs=[
            pl.BlockSpec((1, B), lambda i: (0, 0)),
            pl.BlockSpec((block_rows, B), lambda i: (i, 0)),
        ],
        out_specs=pl.BlockSpec((block_rows, B), lambda i: (i, 0)),
        out_shape=jax.ShapeDtypeStruct((B, B), jnp.float32),
        scratch_shapes=[pltpu.VMEM((1, B), jnp.float32)],
    )(probs_row, logits)


def kernel(logits, candidate_ids, prob_table):
    ids = candidate_ids.reshape(-1).astype(jnp.int32)
    probs = _sc_gather(ids, prob_table)
    return _tc_subtract(logits, probs.reshape(1, B))
